# baseline (speedup 1.0000x reference)
import functools

import jax
import jax.numpy as jnp
from jax import lax
from jax.experimental import pallas as pl
from jax.experimental.pallas import tpu as pltpu
from jax.experimental.pallas import tpu_sc as plsc

N = 10000
E = 320000
D = 128
OUT = 16

NC = 2
NS = 16
NW = NC * NS

CHUNK = 120
CPT = 88
GRP = 8
NGRP = CPT // GRP
NB = 3
EP = NW * CPT * CHUNK
NPAD = 10240
RPT = NPAD // NS
ZROWS = 32

_sc_mesh = plsc.VectorSubcoreMesh(
    core_axis_name="c", subcore_axis_name="s", num_cores=NC, num_subcores=NS)


@functools.partial(
    pl.kernel,
    out_type=jax.ShapeDtypeStruct((NC, NPAD, D), jnp.float32),
    mesh=_sc_mesh,
    scratch_types=[
        pltpu.VMEM((GRP, CHUNK), jnp.int32),
        pltpu.VMEM((CHUNK, D), jnp.float32),
        pltpu.VMEM_SHARED((NPAD, D), jnp.float32),
    ],
)
def _sc_degree(dst_hbm, zeros_hbm, out_hbm, idx_d, ones_v, deg_sh):
  cid = lax.axis_index("c")
  sid = lax.axis_index("s")
  wid = cid * NS + sid

  @pl.loop(0, CHUNK)
  def _(i):
    for c in range(D // 16):
      ones_v[i, pl.ds(c * 16, 16)] = jnp.ones((16,), jnp.float32)

  base = sid * RPT
  pltpu.sync_copy(zeros_hbm, deg_sh.at[pl.ds(base, RPT)])
  plsc.subcore_barrier()

  @pl.loop(0, NGRP)
  def _(g):
    pltpu.sync_copy(dst_hbm.at[pl.ds(wid * NGRP * GRP + g * GRP, GRP)], idx_d)
    for j in range(GRP):
      pltpu.sync_copy(ones_v, deg_sh.at[idx_d.at[j]], add=True)
  plsc.subcore_barrier()

  pltpu.sync_copy(deg_sh.at[pl.ds(base, RPT)],
                  out_hbm.at[cid, pl.ds(base, RPT)])


@functools.partial(
    pl.kernel,
    out_type=jax.ShapeDtypeStruct((NC, NPAD, D), jnp.float32),
    mesh=_sc_mesh,
    scratch_types=[
        pltpu.VMEM((GRP, CHUNK), jnp.int32),
        pltpu.VMEM((GRP, CHUNK), jnp.int32),
        pltpu.VMEM((NB, CHUNK, D), jnp.float32),
        pltpu.VMEM_SHARED((NPAD, D), jnp.float32),
        pltpu.SemaphoreType.DMA,
        pltpu.SemaphoreType.DMA,
        pltpu.SemaphoreType.DMA,
        pltpu.SemaphoreType.DMA,
        pltpu.SemaphoreType.DMA,
        pltpu.SemaphoreType.DMA,
    ],
)
def _sc_aggregate(h_hbm, src_hbm, dst_hbm, zeros_hbm, out_hbm,
                  idx_s, idx_d, rows, agg_sh, g0, g1, g2, s0, s1, s2):
  cid = lax.axis_index("c")
  sid = lax.axis_index("s")
  wid = cid * NS + sid
  gsem = (g0, g1, g2)
  ssem = (s0, s1, s2)

  base = sid * RPT
  pltpu.sync_copy(zeros_hbm, agg_sh.at[pl.ds(base, RPT)])
  plsc.subcore_barrier()

  @pl.loop(0, NGRP)
  def _(g):
    grp_base = wid * CPT + g * GRP
    pltpu.sync_copy(src_hbm.at[pl.ds(grp_base, GRP)], idx_s)
    pltpu.sync_copy(dst_hbm.at[pl.ds(grp_base, GRP)], idx_d)
    for p in range(NB - 1):
      pltpu.async_copy(h_hbm.at[idx_s.at[p]], rows.at[p], gsem[p])
    for j in range(GRP):
      b = j % NB
      pltpu.make_async_copy(h_hbm.at[idx_s.at[j]], rows.at[b],
                            gsem[b]).wait()
      pltpu.async_copy(rows.at[b], agg_sh.at[idx_d.at[j]], ssem[b],
                       add=True)
      nj = j + NB - 1
      if nj < GRP:
        nb = nj % NB
        if nj >= NB:
          pltpu.make_async_copy(rows.at[nb], agg_sh.at[idx_d.at[j]],
                                ssem[nb]).wait()
        pltpu.async_copy(h_hbm.at[idx_s.at[nj]], rows.at[nb], gsem[nb])
    for b in range(NB):
      pltpu.make_async_copy(rows.at[b], agg_sh.at[idx_d.at[0]],
                            ssem[b]).wait()

  plsc.subcore_barrier()
  pltpu.sync_copy(agg_sh.at[pl.ds(base, RPT)],
                  out_hbm.at[cid, pl.ds(base, RPT)])


ROWB = 1000


def _tc_layer_body(h_ref, agg_ref, deg_ref, w_ref, out_ref, sum_ref):
  i = pl.program_id(0)
  a = agg_ref[0] + agg_ref[1]
  d = deg_ref[0] + deg_ref[1]
  x = h_ref[...] + a / jnp.maximum(d, 1.0)
  y = jnp.maximum(jnp.dot(x, w_ref[...], preferred_element_type=jnp.float32),
                  0.0)
  out_ref[...] = y
  s = jnp.broadcast_to(jnp.sum(y, axis=0, keepdims=True), (8, D))
  @pl.when(i == 0)
  def _():
    sum_ref[...] = s
  @pl.when(i > 0)
  def _():
    sum_ref[...] = sum_ref[...] + s


_tc_layer = pl.pallas_call(
    _tc_layer_body,
    grid=(N // ROWB,),
    in_specs=[
        pl.BlockSpec((ROWB, D), lambda i: (i, 0)),
        pl.BlockSpec((NC, ROWB, D), lambda i: (0, i, 0)),
        pl.BlockSpec((NC, ROWB, D), lambda i: (0, i, 0)),
        pl.BlockSpec((D, D), lambda i: (0, 0)),
    ],
    out_specs=[
        pl.BlockSpec((ROWB, D), lambda i: (i, 0)),
        pl.BlockSpec((8, D), lambda i: (0, 0)),
    ],
    out_shape=[
        jax.ShapeDtypeStruct((N, D), jnp.float32),
        jax.ShapeDtypeStruct((8, D), jnp.float32),
    ],
)


def _tc_final_body(s1_ref, s2_ref, s3_ref, wc_ref, o_ref):
  scale = jnp.float32(1.0 / N)
  r = (jnp.dot(s1_ref[0:1], wc_ref[0:D], preferred_element_type=jnp.float32)
       + jnp.dot(s2_ref[0:1], wc_ref[D:2 * D],
                 preferred_element_type=jnp.float32)
       + jnp.dot(s3_ref[0:1], wc_ref[2 * D:3 * D],
                 preferred_element_type=jnp.float32)) * scale
  o_ref[...] = jnp.broadcast_to(r, (8, OUT))


_tc_final = pl.pallas_call(
    _tc_final_body,
    out_shape=jax.ShapeDtypeStruct((8, OUT), jnp.float32),
)


def kernel(x, edge_index, Wp1, Wp2, Wp3, Wm1, Wm2, Wm3, Wc):
  src = edge_index[0]
  dst = edge_index[1]
  pad = EP - E
  src2d = jnp.concatenate(
      [src, jnp.zeros((pad,), jnp.int32)]).reshape(EP // CHUNK, CHUNK)
  dst2d = jnp.concatenate(
      [dst, jnp.full((pad,), N, jnp.int32)]).reshape(EP // CHUNK, CHUNK)
  zeros_rpt = jnp.zeros((RPT, D), jnp.float32)

  degs = _sc_degree(dst2d, zeros_rpt)

  h = x
  for W in (Wp1, Wp2, Wp3):
    aggs = _sc_aggregate(h, src2d, dst2d, zeros_rpt)
    h, _ = _tc_layer(h, aggs, degs, W)

  sums = []
  for W in (Wm1, Wm2, Wm3):
    aggs = _sc_aggregate(h, src2d, dst2d, zeros_rpt)
    h, s = _tc_layer(h, aggs, degs, W)
    sums.append(s)

  out = _tc_final(sums[0], sums[1], sums[2], Wc)
  return out[0]


# speedup vs baseline: 2.0305x; 2.0305x over previous
import functools

import jax
import jax.numpy as jnp
from jax import lax
from jax.experimental import pallas as pl
from jax.experimental.pallas import tpu as pltpu
from jax.experimental.pallas import tpu_sc as plsc

N = 10000
E = 320000
D = 128
OUT = 16

NC = 2
NS = 16
NW = NC * NS

CHUNK = 128
CPT = 80
GRP = 8
NGRP = CPT // GRP
EP = NW * CPT * CHUNK
NPAD = 10240
RPT = NPAD // NS
ZROWS = 32

_sc_mesh = plsc.VectorSubcoreMesh(
    core_axis_name="c", subcore_axis_name="s", num_cores=NC, num_subcores=NS)


@functools.partial(
    pl.kernel,
    out_type=jax.ShapeDtypeStruct((NC, NPAD, D), jnp.float32),
    mesh=_sc_mesh,
    scratch_types=[
        pltpu.VMEM((GRP, CHUNK), jnp.int32),
        pltpu.VMEM((CHUNK, D), jnp.float32),
        pltpu.VMEM_SHARED((NPAD, D), jnp.float32),
    ],
)
def _sc_degree(dst_hbm, zeros_hbm, out_hbm, idx_d, ones_v, deg_sh):
  cid = lax.axis_index("c")
  sid = lax.axis_index("s")
  wid = cid * NS + sid

  @pl.loop(0, CHUNK)
  def _(i):
    for c in range(D // 16):
      ones_v[i, pl.ds(c * 16, 16)] = jnp.ones((16,), jnp.float32)

  base = sid * RPT
  pltpu.sync_copy(zeros_hbm, deg_sh.at[pl.ds(base, RPT)])
  plsc.subcore_barrier()

  @pl.loop(0, NGRP)
  def _(g):
    pltpu.sync_copy(dst_hbm.at[pl.ds(wid * NGRP * GRP + g * GRP, GRP)], idx_d)
    for j in range(GRP):
      pltpu.sync_copy(ones_v, deg_sh.at[idx_d.at[j]], add=True)
  plsc.subcore_barrier()

  pltpu.sync_copy(deg_sh.at[pl.ds(base, RPT)],
                  out_hbm.at[cid, pl.ds(base, RPT)])


@functools.partial(
    pl.kernel,
    out_type=jax.ShapeDtypeStruct((NC, NPAD, D), jnp.float32),
    mesh=_sc_mesh,
    scratch_types=[
        pltpu.VMEM((GRP, CHUNK), jnp.int32),
        pltpu.VMEM((GRP, CHUNK), jnp.int32),
        pltpu.VMEM((2, CHUNK, D), jnp.float32),
        pltpu.VMEM_SHARED((NPAD, D), jnp.float32),
        pltpu.SemaphoreType.DMA,
        pltpu.SemaphoreType.DMA,
    ],
)
def _sc_aggregate(h_hbm, src_hbm, dst_hbm, zeros_hbm, out_hbm,
                  idx_s, idx_d, rows, agg_sh, sem0, sem1):
  cid = lax.axis_index("c")
  sid = lax.axis_index("s")
  wid = cid * NS + sid
  sems = (sem0, sem1)

  base = sid * RPT
  pltpu.sync_copy(zeros_hbm, agg_sh.at[pl.ds(base, RPT)])
  plsc.subcore_barrier()

  @pl.loop(0, NGRP)
  def _(g):
    grp_base = wid * CPT + g * GRP
    pltpu.sync_copy(src_hbm.at[pl.ds(grp_base, GRP)], idx_s)
    pltpu.sync_copy(dst_hbm.at[pl.ds(grp_base, GRP)], idx_d)
    pltpu.async_copy(h_hbm.at[idx_s.at[0]], rows.at[0], sems[0])
    for j in range(GRP):
      b = j % 2
      if j + 1 < GRP:
        pltpu.async_copy(h_hbm.at[idx_s.at[j + 1]], rows.at[1 - b],
                         sems[1 - b])
      pltpu.make_async_copy(h_hbm.at[idx_s.at[j]], rows.at[b],
                            sems[b]).wait()
      pltpu.sync_copy(rows.at[b], agg_sh.at[idx_d.at[j]], add=True)

  plsc.subcore_barrier()
  pltpu.sync_copy(agg_sh.at[pl.ds(base, RPT)],
                  out_hbm.at[cid, pl.ds(base, RPT)])


ROWB = 1000


def _tc_layer_body(h_ref, agg_ref, deg_ref, w_ref, out_ref, sum_ref):
  i = pl.program_id(0)
  a = agg_ref[0] + agg_ref[1]
  d = deg_ref[0] + deg_ref[1]
  x = h_ref[...] + a / jnp.maximum(d, 1.0)
  y = jnp.maximum(jnp.dot(x, w_ref[...], preferred_element_type=jnp.float32),
                  0.0)
  out_ref[...] = y
  s = jnp.broadcast_to(jnp.sum(y, axis=0, keepdims=True), (8, D))
  @pl.when(i == 0)
  def _():
    sum_ref[...] = s
  @pl.when(i > 0)
  def _():
    sum_ref[...] = sum_ref[...] + s


_tc_layer = pl.pallas_call(
    _tc_layer_body,
    grid=(N // ROWB,),
    in_specs=[
        pl.BlockSpec((ROWB, D), lambda i: (i, 0)),
        pl.BlockSpec((NC, ROWB, D), lambda i: (0, i, 0)),
        pl.BlockSpec((NC, ROWB, D), lambda i: (0, i, 0)),
        pl.BlockSpec((D, D), lambda i: (0, 0)),
    ],
    out_specs=[
        pl.BlockSpec((ROWB, D), lambda i: (i, 0)),
        pl.BlockSpec((8, D), lambda i: (0, 0)),
    ],
    out_shape=[
        jax.ShapeDtypeStruct((N, D), jnp.float32),
        jax.ShapeDtypeStruct((8, D), jnp.float32),
    ],
)


def _tc_final_body(s1_ref, s2_ref, s3_ref, wc_ref, o_ref):
  scale = jnp.float32(1.0 / N)
  r = (jnp.dot(s1_ref[0:1], wc_ref[0:D], preferred_element_type=jnp.float32)
       + jnp.dot(s2_ref[0:1], wc_ref[D:2 * D],
                 preferred_element_type=jnp.float32)
       + jnp.dot(s3_ref[0:1], wc_ref[2 * D:3 * D],
                 preferred_element_type=jnp.float32)) * scale
  o_ref[...] = jnp.broadcast_to(r, (8, OUT))


_tc_final = pl.pallas_call(
    _tc_final_body,
    out_shape=jax.ShapeDtypeStruct((8, OUT), jnp.float32),
)


def kernel(x, edge_index, Wp1, Wp2, Wp3, Wm1, Wm2, Wm3, Wc):
  src = edge_index[0]
  dst = edge_index[1]
  pad = EP - E
  src2d = jnp.concatenate(
      [src, jnp.zeros((pad,), jnp.int32)]).reshape(EP // CHUNK, CHUNK)
  dst2d = jnp.concatenate(
      [dst, jnp.full((pad,), N, jnp.int32)]).reshape(EP // CHUNK, CHUNK)
  zeros_rpt = jnp.zeros((RPT, D), jnp.float32)

  degs = _sc_degree(dst2d, zeros_rpt)

  h = x
  for W in (Wp1, Wp2, Wp3):
    aggs = _sc_aggregate(h, src2d, dst2d, zeros_rpt)
    h, _ = _tc_layer(h, aggs, degs, W)

  sums = []
  for W in (Wm1, Wm2, Wm3):
    aggs = _sc_aggregate(h, src2d, dst2d, zeros_rpt)
    h, s = _tc_layer(h, aggs, degs, W)
    sums.append(s)

  out = _tc_final(sums[0], sums[1], sums[2], Wc)
  return out[0]
